# baseline (speedup 1.0000x reference)
import jax
import jax.numpy as jnp
from jax.experimental import pallas as pl
from jax.experimental.pallas import tpu as pltpu

N = 10000
EH = 2500
D = 256
ALPHA = 0.2


def _final_body(u_ref, dv_ref, s_ref, b_ref, o_ref):
    o_ref[...] = u_ref[...] + dv_ref[...] * s_ref[...] + b_ref[...]


def kernel(x, rows, cols, W, a, bias):
    nnz = rows.shape[0]
    ones = jnp.ones(nnz, jnp.float32)
    dv = jax.ops.segment_sum(ones, rows, num_segments=N)
    de = jax.ops.segment_sum(ones, cols, num_segments=EH)
    dv_inv = dv ** -0.5
    de_inv = de ** -1.0
    Xp = x @ W
    xn = Xp * dv_inv[:, None]
    E = jax.ops.segment_sum(xn[rows], cols, num_segments=EH)
    E2 = E * de_inv[:, None]
    F = jax.ops.segment_sum(E2[cols], rows, num_segments=N)
    Yh = F * dv_inv[:, None] + Xp
    a1 = a[:D, 0]
    a2 = a[D:, 0]
    p = Yh @ a1
    q = Yh @ a2
    s = p[rows] + q[cols]
    s = jnp.maximum(s, ALPHA * s)
    es = jnp.exp(s)
    Z = (EH - dv) + jax.ops.segment_sum(es, rows, num_segments=N)
    zinv = 1.0 / Z
    S = (Xp * zinv[:, None]).sum(0)
    w = (es - 1.0) * zinv[rows]
    T = jax.ops.segment_sum(w[:, None] * Xp[rows], cols, num_segments=EH)
    U = jax.ops.segment_sum(T[cols], rows, num_segments=N)

    B = 400
    out = pl.pallas_call(
        _final_body,
        grid=(N // B,),
        in_specs=[
            pl.BlockSpec((B, D), lambda i: (i, 0)),
            pl.BlockSpec((B, 1), lambda i: (i, 0)),
            pl.BlockSpec((1, D), lambda i: (0, 0)),
            pl.BlockSpec((1, D), lambda i: (0, 0)),
        ],
        out_specs=pl.BlockSpec((B, D), lambda i: (i, 0)),
        out_shape=jax.ShapeDtypeStruct((N, D), jnp.float32),
    )(U, dv[:, None], S[None, :], bias[None, :])
    return out


# speedup vs baseline: 9.5320x; 9.5320x over previous
import functools

import jax
import jax.numpy as jnp
from jax import lax
from jax.experimental import pallas as pl
from jax.experimental.pallas import tpu as pltpu
from jax.experimental.pallas import tpu_sc as plsc

N = 10000
EH = 2500
D = 256
ALPHA = 0.2

NC = 2
NS = 16
L = 16
CH = 128

NP = 10240
EP = 4096
NSTR = NP // NS
ESTR = EP // NS

_MESH = plsc.VectorSubcoreMesh(core_axis_name="c", subcore_axis_name="s")
_SC_PARAMS = pltpu.CompilerParams(needs_layout_passes=False)
F32 = jnp.float32
I32 = jnp.int32


def _fill1d(ref, n, val):
    v = jnp.full((L,), val, F32)

    def st(i, _):
        ref[pl.ds(i * L, L)] = v
        return 0

    lax.fori_loop(0, n // L, st, 0)


def _fill2d(ref, rows, val):
    v = jnp.full((L,), val, F32)

    def st(i, _):
        for u in range(CH // L):
            ref[i, pl.ds(u * L, L)] = v
        return 0

    lax.fori_loop(0, rows, st, 0)


def _zero_acc_stripe(gbuf, acc, base, rows):
    _fill2d(gbuf, CH, 0.0)
    for t in range(rows // CH):
        pltpu.sync_copy(gbuf, acc.at[pl.ds(base + t * CH, CH)])


def _make_hist(nchs):
    @functools.partial(
        pl.kernel,
        out_type=[
            jax.ShapeDtypeStruct((NC * NP,), F32),
            jax.ShapeDtypeStruct((NC * EP,), F32),
        ],
        mesh=_MESH,
        compiler_params=_SC_PARAMS,
        scratch_types=[
            pltpu.VMEM((nchs, CH), I32),
            pltpu.VMEM((nchs, CH), I32),
            pltpu.VMEM((CH,), F32),
            pltpu.VMEM((NSTR,), F32),
            pltpu.VMEM_SHARED((NP,), F32),
            pltpu.VMEM_SHARED((EP,), F32),
        ],
    )
    def k(rows_hbm, cols_hbm, dv_hbm, de_hbm, rows_v, cols_v, ones_v, zb1,
          accv, acce):
        c = lax.axis_index("c")
        s = lax.axis_index("s")
        wid = c * NS + s
        _fill1d(ones_v, CH, 1.0)
        _fill1d(zb1, NSTR, 0.0)
        pltpu.sync_copy(zb1, accv.at[pl.ds(s * NSTR, NSTR)])
        pltpu.sync_copy(zb1.at[pl.ds(0, ESTR)], acce.at[pl.ds(s * ESTR, ESTR)])
        pltpu.sync_copy(rows_hbm.at[wid], rows_v)
        pltpu.sync_copy(cols_hbm.at[wid], cols_v)
        plsc.subcore_barrier()

        def step(j, _):
            pltpu.sync_copy(ones_v, accv.at[rows_v.at[j]], add=True)
            pltpu.sync_copy(ones_v, acce.at[cols_v.at[j]], add=True)
            return 0

        lax.fori_loop(0, nchs, step, 0)
        plsc.subcore_barrier()
        pltpu.sync_copy(accv.at[pl.ds(s * NSTR, NSTR)],
                        dv_hbm.at[pl.ds(c * NP + s * NSTR, NSTR)])
        pltpu.sync_copy(acce.at[pl.ds(s * ESTR, ESTR)],
                        de_hbm.at[pl.ds(c * EP + s * ESTR, ESTR)])

    return k


def _make_pass_a(nchv):
    @functools.partial(
        pl.kernel,
        out_type=jax.ShapeDtypeStruct((NC, EP, CH), F32),
        mesh=_MESH,
        compiler_params=_SC_PARAMS,
        scratch_types=[
            pltpu.VMEM((nchv, CH), I32),
            pltpu.VMEM((nchv, CH), I32),
            pltpu.VMEM((CH, CH), F32),
            pltpu.VMEM((ESTR,), F32),
            pltpu.VMEM_SHARED((EP, CH), F32),
            pltpu.SemaphoreType.DMA,
        ],
    )
    def k(xn_hbm, rgo_hbm, cs_hbm, dinv_hbm, out_hbm,
          rg_v, cs_v, gbuf, dinv_v, acc, sem):
        c = lax.axis_index("c")
        s = lax.axis_index("s")
        wid = c * NS + s
        _zero_acc_stripe(gbuf, acc, s * ESTR, ESTR)
        pltpu.sync_copy(rgo_hbm.at[wid], rg_v)
        pltpu.sync_copy(cs_hbm.at[s], cs_v)
        pltpu.sync_copy(dinv_hbm.at[pl.ds(s * ESTR, ESTR)], dinv_v)
        plsc.subcore_barrier()

        def step(j, _):
            pltpu.async_copy(xn_hbm.at[rg_v.at[j]], gbuf, sem).wait()
            pltpu.sync_copy(gbuf, acc.at[cs_v.at[j]], add=True)
            return 0

        lax.fori_loop(0, nchv, step, 0)
        plsc.subcore_barrier()
        for t in range(ESTR // CH):
            base = s * ESTR + t * CH
            pltpu.sync_copy(acc.at[pl.ds(base, CH)], gbuf)

            def scale(r, _):
                b = plsc.load_gather(
                    dinv_v, [jnp.full((L,), t * CH + r, I32)])
                for u in range(CH // L):
                    gbuf[r, pl.ds(u * L, L)] = gbuf[r, pl.ds(u * L, L)] * b
                return 0

            lax.fori_loop(0, CH, scale, 0)
            pltpu.sync_copy(gbuf, out_hbm.at[c, pl.ds(base, CH)])

    return k


def _make_pass_b(nchv, final):
    out_ty = jax.ShapeDtypeStruct((NC, NP, CH), F32)
    scr = [
        pltpu.VMEM((nchv, CH), I32),
        pltpu.VMEM((nchv, CH), I32),
        pltpu.VMEM((CH, CH), F32),
        pltpu.VMEM_SHARED((NP, CH), F32),
        pltpu.SemaphoreType.DMA,
    ]
    if final:
        scr += [
            pltpu.VMEM((NSTR,), F32),
            pltpu.VMEM((8, CH), F32),
            pltpu.VMEM((8, CH), F32),
        ]

    def body(tab_hbm, cgo_hbm, rs_hbm, *rest):
        if final:
            (dv_hbm, svec_hbm, bias_hbm, out_hbm,
             cg_v, rs_v, gbuf, acc, sem, dv_v, s_v, b_v) = rest
        else:
            (out_hbm, cg_v, rs_v, gbuf, acc, sem) = rest
        c = lax.axis_index("c")
        s = lax.axis_index("s")
        wid = c * NS + s
        _zero_acc_stripe(gbuf, acc, s * NSTR, NSTR)
        pltpu.sync_copy(cgo_hbm.at[wid], cg_v)
        pltpu.sync_copy(rs_hbm.at[s], rs_v)
        if final:
            pltpu.sync_copy(dv_hbm.at[pl.ds(s * NSTR, NSTR)], dv_v)
            pltpu.sync_copy(svec_hbm.at[c], s_v)
            pltpu.sync_copy(bias_hbm.at[c], b_v)
        plsc.subcore_barrier()

        def step(j, _):
            pltpu.async_copy(tab_hbm.at[cg_v.at[j]], gbuf, sem).wait()
            pltpu.sync_copy(gbuf, acc.at[rs_v.at[j]], add=True)
            return 0

        lax.fori_loop(0, nchv, step, 0)
        plsc.subcore_barrier()
        if not final:
            pltpu.sync_copy(acc.at[pl.ds(s * NSTR, NSTR)],
                            out_hbm.at[c, pl.ds(s * NSTR, NSTR)])
            return
        sv = [s_v[0, pl.ds(u * L, L)] for u in range(CH // L)]
        bv = [b_v[0, pl.ds(u * L, L)] for u in range(CH // L)]
        for t in range(NSTR // CH):
            base = s * NSTR + t * CH
            pltpu.sync_copy(acc.at[pl.ds(base, CH)], gbuf)

            def asm(r, _):
                db = plsc.load_gather(
                    dv_v, [jnp.full((L,), t * CH + r, I32)])
                for u in range(CH // L):
                    gbuf[r, pl.ds(u * L, L)] = (
                        gbuf[r, pl.ds(u * L, L)] + db * sv[u] + bv[u])
                return 0

            lax.fori_loop(0, CH, asm, 0)
            pltpu.sync_copy(gbuf, out_hbm.at[c, pl.ds(base, CH)])

    return pl.kernel(body, out_type=out_ty, mesh=_MESH, scratch_types=scr,
                     compiler_params=_SC_PARAMS)


def _make_scores(nchs):
    @functools.partial(
        pl.kernel,
        out_type=[
            jax.ShapeDtypeStruct((NC * NS, nchs, CH), F32),
            jax.ShapeDtypeStruct((NC * NP,), F32),
        ],
        mesh=_MESH,
        compiler_params=_SC_PARAMS,
        scratch_types=[
            pltpu.VMEM((nchs, CH), I32),
            pltpu.VMEM((nchs, CH), I32),
            pltpu.VMEM((nchs, CH), F32),
            pltpu.VMEM((NP,), F32),
            pltpu.VMEM((EP,), F32),
            pltpu.VMEM((NSTR,), F32),
            pltpu.VMEM_SHARED((NP,), F32),
        ],
    )
    def k(rows_hbm, cols_hbm, p_hbm, q_hbm, es_hbm, zp_hbm,
          rows_v, cols_v, es_v, p_v, q_v, zb1, accz):
        c = lax.axis_index("c")
        s = lax.axis_index("s")
        wid = c * NS + s
        _fill1d(zb1, NSTR, 0.0)
        pltpu.sync_copy(zb1, accz.at[pl.ds(s * NSTR, NSTR)])
        pltpu.sync_copy(rows_hbm.at[wid], rows_v)
        pltpu.sync_copy(cols_hbm.at[wid], cols_v)
        pltpu.sync_copy(p_hbm, p_v)
        pltpu.sync_copy(q_hbm, q_v)
        plsc.subcore_barrier()

        def step(j, _):
            for u in range(CH // L):
                ri = rows_v[j, pl.ds(u * L, L)]
                ci = cols_v[j, pl.ds(u * L, L)]
                pr = plsc.load_gather(p_v, [ri])
                qr = plsc.load_gather(q_v, [ci])
                sc = pr + qr
                sc = jnp.maximum(sc, ALPHA * sc)
                es_v[j, pl.ds(u * L, L)] = jnp.exp(sc)
            pltpu.sync_copy(es_v.at[j], accz.at[rows_v.at[j]], add=True)
            return 0

        lax.fori_loop(0, nchs, step, 0)
        pltpu.sync_copy(es_v, es_hbm.at[wid])
        plsc.subcore_barrier()
        pltpu.sync_copy(accz.at[pl.ds(s * NSTR, NSTR)],
                        zp_hbm.at[pl.ds(c * NP + s * NSTR, NSTR)])

    return k


def _make_pass_c(nchv):
    @functools.partial(
        pl.kernel,
        out_type=jax.ShapeDtypeStruct((NC, EP, CH), F32),
        mesh=_MESH,
        compiler_params=_SC_PARAMS,
        scratch_types=[
            pltpu.VMEM((nchv, CH), I32),
            pltpu.VMEM((nchv, CH), I32),
            pltpu.VMEM((nchv, CH), F32),
            pltpu.VMEM((CH, CH), F32),
            pltpu.VMEM_SHARED((EP, CH), F32),
            pltpu.SemaphoreType.DMA,
        ],
    )
    def k(xz_hbm, rgo_hbm, cs_hbm, es_hbm, out_hbm,
          rgo_v, cs_v, es_slab, gbuf, acc, sem):
        c = lax.axis_index("c")
        s = lax.axis_index("s")
        wid = c * NS + s
        _zero_acc_stripe(gbuf, acc, s * ESTR, ESTR)
        pltpu.sync_copy(rgo_hbm.at[wid], rgo_v)
        pltpu.sync_copy(cs_hbm.at[s], cs_v)
        pltpu.sync_copy(es_hbm.at[s], es_slab)
        plsc.subcore_barrier()
        one = jnp.full((L,), 1.0, F32)

        def step(j, _):
            pltpu.async_copy(xz_hbm.at[rgo_v.at[j]], gbuf, sem).wait()
            jv = jnp.full((L,), j, I32)

            def rowmul(r, _):
                rv = jnp.full((L,), r, I32)
                ev = plsc.load_gather(es_slab, [jv, rv])
                wb = ev - one
                for u in range(CH // L):
                    gbuf[r, pl.ds(u * L, L)] = gbuf[r, pl.ds(u * L, L)] * wb
                return 0

            lax.fori_loop(0, CH, rowmul, 0)
            pltpu.sync_copy(gbuf, acc.at[cs_v.at[j]], add=True)
            return 0

        lax.fori_loop(0, nchv, step, 0)
        plsc.subcore_barrier()
        pltpu.sync_copy(acc.at[pl.ds(s * ESTR, ESTR)],
                        out_hbm.at[c, pl.ds(s * ESTR, ESTR)])

    return k


def _k2_body(x_ref, w_ref, dvi_ref, xp_ref, xn_ref):
    xp = jnp.dot(x_ref[...], w_ref[...], preferred_element_type=F32)
    xp_ref[0] = xp
    xn_ref[0] = xp * dvi_ref[...]


def _k5_body(f_ref, xp_ref, dvi_ref, am_ref, pq_ref):
    yh = f_ref[0] * dvi_ref[...] + xp_ref[0]
    part = jnp.dot(yh, am_ref[0], preferred_element_type=F32)

    @pl.when(pl.program_id(0) == 0)
    def _():
        pq_ref[...] = part

    @pl.when(pl.program_id(0) != 0)
    def _():
        pq_ref[...] = pq_ref[...] + part


def _k7_body(xp_ref, dv_ref, zs_ref, xz_ref, s_ref, zi_ref):
    zi = 1.0 / (float(EH) - dv_ref[...] + zs_ref[...])
    xz = xp_ref[0] * zi
    xz_ref[0] = xz
    zi_ref[...] = zi

    @pl.when(pl.program_id(1) == 0)
    def _():
        s_ref[...] = jnp.zeros_like(s_ref)

    part = jnp.sum(xz, axis=0, keepdims=True)
    s_ref[...] = s_ref[...] + jnp.broadcast_to(part, (1, 8, CH))


def kernel(x, rows, cols, W, a, bias):
    nnz = rows.shape[0]
    LP = -(-nnz // 4096) * 4096
    pad = LP - nnz
    nchv = LP // (NS * CH)
    nchs = LP // (NC * NS * CH)

    rows_g = jnp.pad(rows, (0, pad))
    rows_s = jnp.pad(rows, (0, pad), constant_values=N)
    cols_g = jnp.pad(cols, (0, pad))
    cols_s = jnp.pad(cols, (0, pad), constant_values=EH)
    rgo = jnp.stack([rows_g, rows_g + NP]).reshape(NC, NS, nchv, CH)
    rgo = rgo.reshape(NC * NS, nchv, CH)
    cgo = jnp.stack([cols_g, cols_g + EP]).reshape(NC, NS, nchv, CH)
    cgo = cgo.reshape(NC * NS, nchv, CH)
    rg16 = rows_g.reshape(NS, nchv, CH)
    rs16 = rows_s.reshape(NS, nchv, CH)
    cs16 = cols_s.reshape(NS, nchv, CH)
    rs32 = rows_s.reshape(NC * NS, nchs, CH)
    cs32 = cols_s.reshape(NC * NS, nchs, CH)

    dvp, dep = _make_hist(nchs)(rs32, cs32)
    dvp = dvp.reshape(NC, NP)
    dep = dep.reshape(NC, EP)
    dv_pad = dvp[0] + dvp[1]
    de_pad = dep[0] + dep[1]
    dvi_pad = jnp.maximum(dv_pad, 1.0) ** -0.5
    dinv_pad = 1.0 / jnp.maximum(de_pad, 1.0)

    x_pad = jnp.pad(x, ((0, NP - N), (0, 0)))
    B = 640
    xp_h, xn_h = pl.pallas_call(
        _k2_body,
        grid=(NC, NP // B),
        in_specs=[
            pl.BlockSpec((B, D), lambda h, i: (i, 0)),
            pl.BlockSpec((D, CH), lambda h, i: (0, h)),
            pl.BlockSpec((B, 1), lambda h, i: (i, 0)),
        ],
        out_specs=[
            pl.BlockSpec((1, B, CH), lambda h, i: (h, i, 0)),
            pl.BlockSpec((1, B, CH), lambda h, i: (h, i, 0)),
        ],
        out_shape=[
            jax.ShapeDtypeStruct((NC, NP, CH), F32),
            jax.ShapeDtypeStruct((NC, NP, CH), F32),
        ],
    )(x_pad, W, dvi_pad[:, None])

    e2 = _make_pass_a(nchv)(xn_h.reshape(NC * NP, CH), rgo, cs16, dinv_pad)

    f_h = _make_pass_b(nchv, final=False)(e2.reshape(NC * EP, CH), cgo, rs16)

    a1h = a[:D, 0].reshape(NC, CH)
    a2h = a[D:, 0].reshape(NC, CH)
    am = jnp.concatenate(
        [a1h[:, :, None], a2h[:, :, None], jnp.zeros((NC, CH, CH - 2), F32)],
        axis=2)
    pq = pl.pallas_call(
        _k5_body,
        grid=(NC,),
        in_specs=[
            pl.BlockSpec((1, NP, CH), lambda h: (h, 0, 0)),
            pl.BlockSpec((1, NP, CH), lambda h: (h, 0, 0)),
            pl.BlockSpec((NP, 1), lambda h: (0, 0)),
            pl.BlockSpec((1, CH, CH), lambda h: (h, 0, 0)),
        ],
        out_specs=pl.BlockSpec((NP, CH), lambda h: (0, 0)),
        out_shape=jax.ShapeDtypeStruct((NP, CH), F32),
    )(f_h, xp_h, dvi_pad[:, None], am)
    p_pad = pq[:, 0]
    q_pad = pq[:EP, 1]

    es, zp = _make_scores(nchs)(rs32, cs32, p_pad, q_pad)
    zp = zp.reshape(NC, NP)
    zsum_pad = zp[0] + zp[1]

    xz_h, svec, zinv = pl.pallas_call(
        _k7_body,
        grid=(NC, NP // B),
        in_specs=[
            pl.BlockSpec((1, B, CH), lambda h, i: (h, i, 0)),
            pl.BlockSpec((B, 1), lambda h, i: (i, 0)),
            pl.BlockSpec((B, 1), lambda h, i: (i, 0)),
        ],
        out_specs=[
            pl.BlockSpec((1, B, CH), lambda h, i: (h, i, 0)),
            pl.BlockSpec((1, 8, CH), lambda h, i: (h, 0, 0)),
            pl.BlockSpec((B, 1), lambda h, i: (i, 0)),
        ],
        out_shape=[
            jax.ShapeDtypeStruct((NC, NP, CH), F32),
            jax.ShapeDtypeStruct((NC, 8, CH), F32),
            jax.ShapeDtypeStruct((NP, 1), F32),
        ],
    )(xp_h, dv_pad[:, None], zsum_pad[:, None])

    es16 = es.reshape(NS, nchv, CH)
    t_h = _make_pass_c(nchv)(xz_h.reshape(NC * NP, CH), rgo, cs16, es16)

    bias2 = jnp.broadcast_to(bias.reshape(NC, 1, CH), (NC, 8, CH))
    u_h = _make_pass_b(nchv, final=True)(
        t_h.reshape(NC * EP, CH), cgo, rs16, dv_pad, svec, bias2)

    return jnp.concatenate([u_h[0, :N], u_h[1, :N]], axis=1)


# speedup vs baseline: 12.3780x; 1.2986x over previous
import functools

import jax
import jax.numpy as jnp
from jax import lax
from jax.experimental import pallas as pl
from jax.experimental.pallas import tpu as pltpu
from jax.experimental.pallas import tpu_sc as plsc

N = 10000
EH = 2500
D = 256
ALPHA = 0.2

NC = 2
NS = 16
L = 16
CH = 128
CB = 128

NP = 10240
EP = 4096
NSTR = NP // NS
ESTR = EP // NS

_MESH = plsc.VectorSubcoreMesh(core_axis_name="c", subcore_axis_name="s")
_SC_PARAMS = pltpu.CompilerParams(needs_layout_passes=False)
F32 = jnp.float32
I32 = jnp.int32


def _fill1d(ref, n, val):
    v = jnp.full((L,), val, F32)

    def st(i, _):
        ref[pl.ds(i * L, L)] = v
        return 0

    lax.fori_loop(0, n // L, st, 0)


def _fill2d(ref, rows, val):
    v = jnp.full((L,), val, F32)

    def st(i, _):
        for u in range(CH // L):
            ref[i, pl.ds(u * L, L)] = v
        return 0

    lax.fori_loop(0, rows, st, 0)


def _zero_acc_stripe(gbuf, grows, acc, base, rows):
    _fill2d(gbuf, grows, 0.0)
    for t in range(rows // grows):
        pltpu.sync_copy(gbuf, acc.at[pl.ds(base + t * grows, grows)])


def _offset_idx(raw_ref, off_ref, nch, off):
    def st(i, _):
        j = i // (CH // L)
        u = i % (CH // L)
        off_ref[j, pl.ds(u * L, L)] = raw_ref[j, pl.ds(u * L, L)] + off
        return 0

    lax.fori_loop(0, nch * (CH // L), st, 0)


def _pipelined_chunks(nch, tab_hbm, idx_ref, buf0, buf1, sem0, sem1, consume):
    pltpu.async_copy(tab_hbm.at[idx_ref.at[0]], buf0, sem0)

    def step(j2, _):
        j = 2 * j2
        pltpu.async_copy(tab_hbm.at[idx_ref.at[j + 1]], buf1, sem1)
        pltpu.make_async_copy(tab_hbm.at[idx_ref.at[j]], buf0, sem0).wait()
        consume(j, buf0)
        nx = jnp.minimum(j + 2, nch - 1)
        pltpu.async_copy(tab_hbm.at[idx_ref.at[nx]], buf0, sem0)
        pltpu.make_async_copy(
            tab_hbm.at[idx_ref.at[j + 1]], buf1, sem1).wait()
        consume(j + 1, buf1)
        return 0

    lax.fori_loop(0, nch // 2, step, 0)
    pltpu.make_async_copy(tab_hbm.at[idx_ref.at[nch - 1]], buf0, sem0).wait()


def _make_hist(nchs):
    @functools.partial(
        pl.kernel,
        out_type=[
            jax.ShapeDtypeStruct((NC * NP,), F32),
            jax.ShapeDtypeStruct((NC * EP,), F32),
        ],
        mesh=_MESH,
        compiler_params=_SC_PARAMS,
        scratch_types=[
            pltpu.VMEM((nchs, CH), I32),
            pltpu.VMEM((nchs, CH), I32),
            pltpu.VMEM((CH,), F32),
            pltpu.VMEM((NSTR,), F32),
            pltpu.VMEM_SHARED((NP,), F32),
            pltpu.VMEM_SHARED((EP,), F32),
        ],
    )
    def k(rows_hbm, cols_hbm, dv_hbm, de_hbm, rows_v, cols_v, ones_v, zb1,
          accv, acce):
        c = lax.axis_index("c")
        s = lax.axis_index("s")
        wid = c * NS + s
        _fill1d(ones_v, CH, 1.0)
        _fill1d(zb1, NSTR, 0.0)
        pltpu.sync_copy(zb1, accv.at[pl.ds(s * NSTR, NSTR)])
        pltpu.sync_copy(zb1.at[pl.ds(0, ESTR)], acce.at[pl.ds(s * ESTR, ESTR)])
        pltpu.sync_copy(rows_hbm.at[wid], rows_v)
        pltpu.sync_copy(cols_hbm.at[wid], cols_v)
        plsc.subcore_barrier()

        def step(j, _):
            pltpu.sync_copy(ones_v, accv.at[rows_v.at[j]], add=True)
            pltpu.sync_copy(ones_v, acce.at[cols_v.at[j]], add=True)
            return 0

        lax.fori_loop(0, nchs, step, 0)
        plsc.subcore_barrier()
        pltpu.sync_copy(accv.at[pl.ds(s * NSTR, NSTR)],
                        dv_hbm.at[pl.ds(c * NP + s * NSTR, NSTR)])
        pltpu.sync_copy(acce.at[pl.ds(s * ESTR, ESTR)],
                        de_hbm.at[pl.ds(c * EP + s * ESTR, ESTR)])

    return k


def _make_pass_a(nchv):
    @functools.partial(
        pl.kernel,
        out_type=jax.ShapeDtypeStruct((NC, EP, CH), F32),
        mesh=_MESH,
        compiler_params=_SC_PARAMS,
        scratch_types=[
            pltpu.VMEM((nchv, CH), I32),
            pltpu.VMEM((nchv, CH), I32),
            pltpu.VMEM((CH, CH), F32),
            pltpu.VMEM((CH, CH), F32),
            pltpu.VMEM((ESTR,), F32),
            pltpu.VMEM_SHARED((EP, CH), F32),
            pltpu.SemaphoreType.DMA,
            pltpu.SemaphoreType.DMA,
        ],
    )
    def k(xn_hbm, rgo_hbm, cs_hbm, dinv_hbm, out_hbm,
          rg_v, cs_v, gbuf0, gbuf1, dinv_v, acc, sem0, sem1):
        c = lax.axis_index("c")
        s = lax.axis_index("s")
        wid = c * NS + s
        _zero_acc_stripe(gbuf0, CH, acc, s * ESTR, ESTR)
        pltpu.sync_copy(rgo_hbm.at[wid], rg_v)
        pltpu.sync_copy(cs_hbm.at[s], cs_v)
        pltpu.sync_copy(dinv_hbm.at[pl.ds(s * ESTR, ESTR)], dinv_v)
        plsc.subcore_barrier()

        def consume(j, buf):
            pltpu.sync_copy(buf, acc.at[cs_v.at[j]], add=True)

        _pipelined_chunks(nchv, xn_hbm, rg_v, gbuf0, gbuf1, sem0, sem1,
                          consume)
        plsc.subcore_barrier()
        for t in range(ESTR // CH):
            base = s * ESTR + t * CH
            pltpu.sync_copy(acc.at[pl.ds(base, CH)], gbuf0)

            def scale(r, _):
                b = plsc.load_gather(
                    dinv_v, [jnp.full((L,), t * CH + r, I32)])
                for u in range(CH // L):
                    gbuf0[r, pl.ds(u * L, L)] = gbuf0[r, pl.ds(u * L, L)] * b
                return 0

            lax.fori_loop(0, CH, scale, 0)
            pltpu.sync_copy(gbuf0, out_hbm.at[c, pl.ds(base, CH)])

    return k


def _make_pass_b(nchb, final):
    out_ty = jax.ShapeDtypeStruct((NC, NP, CH), F32)
    scr = [
        pltpu.VMEM((nchb, CB), I32),
        pltpu.VMEM((nchb, CB), I32),
        pltpu.VMEM((CB, CH), F32),
        pltpu.VMEM_SHARED((NP, CH), F32),
        pltpu.SemaphoreType.DMA,
    ]
    if final:
        scr += [
            pltpu.VMEM((NSTR,), F32),
            pltpu.VMEM((CH,), F32),
            pltpu.VMEM((CH,), F32),
        ]

    def body(tab_hbm, cgo_hbm, rs_hbm, *rest):
        if final:
            (dv_hbm, svec_hbm, bias_hbm, out_hbm,
             cg_v, rs_v, gbuf0, acc, sem0, dv_v, s_v, b_v) = rest
        else:
            (out_hbm, cg_v, rs_v, gbuf0, acc, sem0) = rest
        c = lax.axis_index("c")
        s = lax.axis_index("s")
        wid = c * NS + s
        _zero_acc_stripe(gbuf0, CB, acc, s * NSTR, NSTR)
        pltpu.sync_copy(cgo_hbm.at[wid], cg_v)
        pltpu.sync_copy(rs_hbm.at[s], rs_v)
        if final:
            pltpu.sync_copy(dv_hbm.at[pl.ds(s * NSTR, NSTR)], dv_v)
            pltpu.sync_copy(svec_hbm.at[pl.ds(c * 8 * CH, CH)], s_v)
            pltpu.sync_copy(bias_hbm.at[pl.ds(c * CH, CH)], b_v)
        plsc.subcore_barrier()

        def step(j, _):
            pltpu.async_copy(tab_hbm.at[cg_v.at[j]], gbuf0, sem0).wait()
            pltpu.sync_copy(gbuf0, acc.at[rs_v.at[j]], add=True)
            return 0

        lax.fori_loop(0, nchb, step, 0)
        plsc.subcore_barrier()
        if not final:
            pltpu.sync_copy(acc.at[pl.ds(s * NSTR, NSTR)],
                            out_hbm.at[c, pl.ds(s * NSTR, NSTR)])
            return
        sv = [s_v[pl.ds(u * L, L)] for u in range(CH // L)]
        bv = [b_v[pl.ds(u * L, L)] for u in range(CH // L)]
        for t in range(NSTR // CB):
            base = s * NSTR + t * CB
            pltpu.sync_copy(acc.at[pl.ds(base, CB)], gbuf0)

            def asm(r, _):
                db = plsc.load_gather(
                    dv_v, [jnp.full((L,), t * CB + r, I32)])
                for u in range(CH // L):
                    gbuf0[r, pl.ds(u * L, L)] = (
                        gbuf0[r, pl.ds(u * L, L)] + db * sv[u] + bv[u])
                return 0

            lax.fori_loop(0, CB, asm, 0)
            pltpu.sync_copy(gbuf0, out_hbm.at[c, pl.ds(base, CB)])

    return pl.kernel(body, out_type=out_ty, mesh=_MESH, scratch_types=scr,
                     compiler_params=_SC_PARAMS)


def _make_scores(nchs):
    @functools.partial(
        pl.kernel,
        out_type=[
            jax.ShapeDtypeStruct((NC * NS, nchs, CH), F32),
            jax.ShapeDtypeStruct((NC * NP,), F32),
        ],
        mesh=_MESH,
        compiler_params=_SC_PARAMS,
        scratch_types=[
            pltpu.VMEM((nchs, CH), I32),
            pltpu.VMEM((nchs, CH), I32),
            pltpu.VMEM((nchs, CH), F32),
            pltpu.VMEM((NP,), F32),
            pltpu.VMEM((EP,), F32),
            pltpu.VMEM((NSTR,), F32),
            pltpu.VMEM_SHARED((NP,), F32),
        ],
    )
    def k(rows_hbm, cols_hbm, p_hbm, q_hbm, es_hbm, zp_hbm,
          rows_v, cols_v, es_v, p_v, q_v, zb1, accz):
        c = lax.axis_index("c")
        s = lax.axis_index("s")
        wid = c * NS + s
        _fill1d(zb1, NSTR, 0.0)
        pltpu.sync_copy(zb1, accz.at[pl.ds(s * NSTR, NSTR)])
        pltpu.sync_copy(rows_hbm.at[wid], rows_v)
        pltpu.sync_copy(cols_hbm.at[wid], cols_v)
        pltpu.sync_copy(p_hbm, p_v)
        pltpu.sync_copy(q_hbm, q_v)
        plsc.subcore_barrier()

        def step(j, _):
            for u in range(CH // L):
                ri = rows_v[j, pl.ds(u * L, L)]
                ci = cols_v[j, pl.ds(u * L, L)]
                pr = plsc.load_gather(p_v, [ri])
                qr = plsc.load_gather(q_v, [ci])
                sc = pr + qr
                sc = jnp.maximum(sc, ALPHA * sc)
                es_v[j, pl.ds(u * L, L)] = jnp.exp(sc)
            pltpu.sync_copy(es_v.at[j], accz.at[rows_v.at[j]], add=True)
            return 0

        lax.fori_loop(0, nchs, step, 0)
        pltpu.sync_copy(es_v, es_hbm.at[wid])
        plsc.subcore_barrier()
        pltpu.sync_copy(accz.at[pl.ds(s * NSTR, NSTR)],
                        zp_hbm.at[pl.ds(c * NP + s * NSTR, NSTR)])

    return k


def _make_pass_c(nchv):
    @functools.partial(
        pl.kernel,
        out_type=jax.ShapeDtypeStruct((NC, EP, CH), F32),
        mesh=_MESH,
        compiler_params=_SC_PARAMS,
        scratch_types=[
            pltpu.VMEM((nchv, CH), I32),
            pltpu.VMEM((nchv, CH), I32),
            pltpu.VMEM((nchv, CH), F32),
            pltpu.VMEM((CH, CH), F32),
            pltpu.VMEM((CH, CH), F32),
            pltpu.VMEM_SHARED((EP, CH), F32),
            pltpu.SemaphoreType.DMA,
            pltpu.SemaphoreType.DMA,
        ],
    )
    def k(xz_hbm, rgo_hbm, cs_hbm, es_hbm, out_hbm,
          rgo_v, cs_v, es_slab, gbuf0, gbuf1, acc, sem0, sem1):
        c = lax.axis_index("c")
        s = lax.axis_index("s")
        wid = c * NS + s
        _zero_acc_stripe(gbuf0, CH, acc, s * ESTR, ESTR)
        pltpu.sync_copy(rgo_hbm.at[wid], rgo_v)
        pltpu.sync_copy(cs_hbm.at[s], cs_v)
        pltpu.sync_copy(es_hbm.at[s], es_slab)
        plsc.subcore_barrier()
        one = jnp.full((L,), 1.0, F32)

        def consume(j, buf):
            jv = jnp.full((L,), j, I32)

            def rowmul(r, _):
                rv = jnp.full((L,), r, I32)
                ev = plsc.load_gather(es_slab, [jv, rv])
                wb = ev - one
                for u in range(CH // L):
                    buf[r, pl.ds(u * L, L)] = buf[r, pl.ds(u * L, L)] * wb
                return 0

            lax.fori_loop(0, CH, rowmul, 0, unroll=2)
            pltpu.sync_copy(buf, acc.at[cs_v.at[j]], add=True)

        _pipelined_chunks(nchv, xz_hbm, rgo_v, gbuf0, gbuf1, sem0, sem1,
                          consume)
        plsc.subcore_barrier()
        pltpu.sync_copy(acc.at[pl.ds(s * ESTR, ESTR)],
                        out_hbm.at[c, pl.ds(s * ESTR, ESTR)])

    return k


def _k2_body(x_ref, w_ref, dvi_ref, xp_ref, xn_ref):
    xp = jnp.dot(x_ref[...], w_ref[...], preferred_element_type=F32)
    xp_ref[0] = xp
    xn_ref[0] = xp * dvi_ref[...]


def _k5_body(f_ref, xp_ref, dvi_ref, am_ref, pq_ref):
    yh = f_ref[0] * dvi_ref[...] + xp_ref[0]
    part = jnp.dot(yh, am_ref[0], preferred_element_type=F32)

    @pl.when(pl.program_id(0) == 0)
    def _():
        pq_ref[...] = part

    @pl.when(pl.program_id(0) != 0)
    def _():
        pq_ref[...] = pq_ref[...] + part


def _k7_body(xp_ref, dv_ref, zs_ref, xz_ref, s_ref):
    zi = 1.0 / (float(EH) - dv_ref[...] + zs_ref[...])
    xz = xp_ref[0] * zi
    xz_ref[0] = xz

    @pl.when(pl.program_id(1) == 0)
    def _():
        s_ref[...] = jnp.zeros_like(s_ref)

    part = jnp.sum(xz, axis=0, keepdims=True)
    s_ref[...] = s_ref[...] + jnp.broadcast_to(part, (1, 8, CH))


def kernel(x, rows, cols, W, a, bias):
    nnz = rows.shape[0]
    LP = -(-nnz // 4096) * 4096
    pad = LP - nnz
    nchv = LP // (NS * CH)
    nchb = LP // (NS * CB)
    nchs = LP // (NC * NS * CH)

    rows_g = jnp.pad(rows, (0, pad))
    rows_s = jnp.pad(rows, (0, pad), constant_values=N)
    cols_g = jnp.pad(cols, (0, pad))
    cols_s = jnp.pad(cols, (0, pad), constant_values=EH)
    rgo = jnp.stack([rows_g, rows_g + NP]).reshape(NC * NS, nchv, CH)
    cgo = jnp.stack([cols_g, cols_g + EP]).reshape(NC * NS, nchb, CB)
    cs16 = cols_s.reshape(NS, nchv, CH)
    rs16 = rows_s.reshape(NS, nchb, CB)
    rs32 = rows_s.reshape(NC * NS, nchs, CH)
    cs32 = cols_s.reshape(NC * NS, nchs, CH)

    dvp, dep = _make_hist(nchs)(rs32, cs32)
    dvp = dvp.reshape(NC, NP)
    dep = dep.reshape(NC, EP)
    dv_pad = dvp[0] + dvp[1]
    de_pad = dep[0] + dep[1]
    dvi_pad = jnp.maximum(dv_pad, 1.0) ** -0.5
    dinv_pad = 1.0 / jnp.maximum(de_pad, 1.0)

    x_pad = jnp.pad(x, ((0, NP - N), (0, 0)))
    B = 640
    xp_h, xn_h = pl.pallas_call(
        _k2_body,
        grid=(NC, NP // B),
        in_specs=[
            pl.BlockSpec((B, D), lambda h, i: (i, 0)),
            pl.BlockSpec((D, CH), lambda h, i: (0, h)),
            pl.BlockSpec((B, 1), lambda h, i: (i, 0)),
        ],
        out_specs=[
            pl.BlockSpec((1, B, CH), lambda h, i: (h, i, 0)),
            pl.BlockSpec((1, B, CH), lambda h, i: (h, i, 0)),
        ],
        out_shape=[
            jax.ShapeDtypeStruct((NC, NP, CH), F32),
            jax.ShapeDtypeStruct((NC, NP, CH), F32),
        ],
    )(x_pad, W, dvi_pad[:, None])

    e2 = _make_pass_a(nchv)(xn_h.reshape(NC * NP, CH), rgo, cs16, dinv_pad)

    f_h = _make_pass_b(nchb, final=False)(e2.reshape(NC * EP, CH), cgo, rs16)

    a1h = a[:D, 0].reshape(NC, CH)
    a2h = a[D:, 0].reshape(NC, CH)
    am = jnp.concatenate(
        [a1h[:, :, None], a2h[:, :, None], jnp.zeros((NC, CH, CH - 2), F32)],
        axis=2)
    pq = pl.pallas_call(
        _k5_body,
        grid=(NC,),
        in_specs=[
            pl.BlockSpec((1, NP, CH), lambda h: (h, 0, 0)),
            pl.BlockSpec((1, NP, CH), lambda h: (h, 0, 0)),
            pl.BlockSpec((NP, 1), lambda h: (0, 0)),
            pl.BlockSpec((1, CH, CH), lambda h: (h, 0, 0)),
        ],
        out_specs=pl.BlockSpec((NP, CH), lambda h: (0, 0)),
        out_shape=jax.ShapeDtypeStruct((NP, CH), F32),
    )(f_h, xp_h, dvi_pad[:, None], am)
    p_pad = pq[:, 0]
    q_pad = pq[:EP, 1]

    es, zp = _make_scores(nchs)(rs32, cs32, p_pad, q_pad)
    zp = zp.reshape(NC, NP)
    zsum_pad = zp[0] + zp[1]

    xz_h, svec = pl.pallas_call(
        _k7_body,
        grid=(NC, NP // B),
        in_specs=[
            pl.BlockSpec((1, B, CH), lambda h, i: (h, i, 0)),
            pl.BlockSpec((B, 1), lambda h, i: (i, 0)),
            pl.BlockSpec((B, 1), lambda h, i: (i, 0)),
        ],
        out_specs=[
            pl.BlockSpec((1, B, CH), lambda h, i: (h, i, 0)),
            pl.BlockSpec((1, 8, CH), lambda h, i: (h, 0, 0)),
        ],
        out_shape=[
            jax.ShapeDtypeStruct((NC, NP, CH), F32),
            jax.ShapeDtypeStruct((NC, 8, CH), F32),
        ],
    )(xp_h, dv_pad[:, None], zsum_pad[:, None])

    es16 = es.reshape(NS, nchv, CH)
    t_h = _make_pass_c(nchv)(xz_h.reshape(NC * NP, CH), rgo, cs16, es16)

    u_h = _make_pass_b(nchb, final=True)(
        t_h.reshape(NC * EP, CH), cgo, rs16, dv_pad,
        svec.reshape(NC * 8 * CH), bias)

    return jnp.concatenate([u_h[0, :N], u_h[1, :N]], axis=1)
